# baseline (speedup 1.0000x reference)
import functools

import jax
import jax.numpy as jnp
from jax import lax
from jax.experimental import pallas as pl
from jax.experimental.pallas import tpu as pltpu
from jax.experimental.pallas import tpu_sc as plsc

B = 16384
D_DRV, D_WEEK, D_TIME = 16, 3, 8
D_OUT = D_DRV + D_WEEK + D_TIME + 1
V_DRV, V_WEEK, V_TIME = 24000, 7, 1440
SUP = 128
RPS = SUP // D_DRV
NC, NS = 2, 16
NW = NC * NS
BPW = B // NW
CHUNK = 128
NCHUNK = BPW // CHUNK
L = 16
GPC = CHUNK // L
SLABS_PER_TILE = 12
NSLAB = NS * SLABS_PER_TILE
V_PAD = NSLAB * CHUNK
SUP_PER_SC = V_PAD // RPS

_mesh = plsc.VectorSubcoreMesh(core_axis_name="c", subcore_axis_name="s")


@functools.partial(
    pl.kernel,
    mesh=_mesh,
    out_type=(jax.ShapeDtypeStruct((D_OUT, B), jnp.float32),
              jax.ShapeDtypeStruct((NC * SUP_PER_SC, SUP), jnp.float32)),
    compiler_params=pltpu.CompilerParams(needs_layout_passes=False),
    scratch_types=[
        pltpu.VMEM((BPW,), jnp.int32),
        pltpu.VMEM((BPW,), jnp.int32),
        pltpu.VMEM((BPW,), jnp.int32),
        pltpu.VMEM((BPW,), jnp.int32),
        pltpu.VMEM((BPW,), jnp.float32),
        pltpu.VMEM((BPW, SUP), jnp.float32),
        pltpu.VMEM((V_WEEK * D_WEEK,), jnp.float32),
        pltpu.VMEM((V_TIME * D_TIME,), jnp.float32),
        pltpu.VMEM((2, D_OUT, CHUNK), jnp.float32),
        pltpu.VMEM((2, D_DRV, CHUNK), jnp.float32),
        pltpu.VMEM((2, CHUNK // RPS, SUP), jnp.float32),
        [pltpu.SemaphoreType.DMA] * NCHUNK,
        [pltpu.SemaphoreType.DMA] * 2,
        pltpu.SemaphoreType.DMA,
        [pltpu.SemaphoreType.DMA] * 2,
        [pltpu.SemaphoreType.DMA] * 2,
    ],
)
def _embed_concat(drv_id, week_id, time_id, dist, wt_pad, w_week, w_time,
                  out, tbl, drv_idx, drv_sup, week_idx, time_idx, dist_v,
                  g_buf, week_tbl, time_tbl, out_buf, slab, srow,
                  gsems, osems, ssem, rsems, wsems):
    cc = lax.axis_index("c")
    sid = lax.axis_index("s")
    wid = sid * NC + cc
    base = wid * BPW
    sc_row0 = cc * SUP_PER_SC

    stagers = [
        pltpu.async_copy(week_id.at[pl.ds(base, BPW)], week_idx, ssem),
        pltpu.async_copy(time_id.at[pl.ds(base, BPW)], time_idx, ssem),
        pltpu.async_copy(dist.at[pl.ds(base, BPW)], dist_v, ssem),
        pltpu.async_copy(w_week, week_tbl, ssem),
        pltpu.async_copy(w_time, time_tbl, ssem),
    ]
    pltpu.sync_copy(drv_id.at[pl.ds(base, BPW)], drv_idx)

    iota = lax.iota(jnp.int32, L)
    cols = [jnp.full((L,), c, jnp.int32) for c in range(D_OUT)]

    def sup_body(g, carry):
        v = drv_idx[pl.ds(g * L, L)]
        drv_sup[pl.ds(g * L, L)] = lax.shift_right_logical(v, 3) + sc_row0
        return carry

    lax.fori_loop(0, BPW // L, sup_body, 0)

    def slab_read(i, buf):
        b = sid * SLABS_PER_TILE + i
        col0 = pl.multiple_of(b * CHUNK, CHUNK)
        return pltpu.async_copy(wt_pad.at[:, pl.ds(col0, CHUNK)],
                                slab.at[buf], rsems[buf])

    reads = [slab_read(0, 0), slab_read(1, 1)]
    writes = [None, None]
    for i in range(SLABS_PER_TILE):
        bi = i % 2
        reads[bi].wait()
        if writes[bi] is not None:
            writes[bi].wait()

        def rep_body(k, carry):
            for j in range(RPS):
                v = plsc.load_gather(slab.at[bi], [iota, cols[0] + (k * RPS + j)])
                srow[bi, k, pl.ds(j * D_DRV, D_DRV)] = v
            return carry

        lax.fori_loop(0, CHUNK // RPS, rep_body, 0, unroll=4)
        b = sid * SLABS_PER_TILE + i
        row0 = pl.multiple_of(sc_row0 + b * (CHUNK // RPS), CHUNK // RPS)
        writes[bi] = pltpu.async_copy(
            srow.at[bi], tbl.at[pl.ds(row0, CHUNK // RPS)], wsems[bi])
        if i + 2 < SLABS_PER_TILE:
            reads[bi] = slab_read(i + 2, bi)
    for w in writes:
        w.wait()
    plsc.subcore_barrier()

    gathers = [
        pltpu.async_copy(tbl.at[drv_sup.at[pl.ds(j * CHUNK, CHUNK)]],
                         g_buf.at[pl.ds(j * CHUNK, CHUNK)], gsems[j])
        for j in range(NCHUNK)
    ]
    for s in stagers:
        s.wait()

    out_writes = [None, None]
    for r in range(NCHUNK):
        buf = out_buf.at[r % 2]
        if out_writes[r % 2] is not None:
            out_writes[r % 2].wait()
        for g in range(GPC):
            rows = g * L + iota
            src = r * CHUNK + g * L
            widx = week_idx[pl.ds(src, L)] * D_WEEK
            for j in range(D_WEEK):
                v = plsc.load_gather(week_tbl, [widx + j])
                plsc.store_scatter(buf, [cols[D_DRV + j], rows], v)
            tidx = time_idx[pl.ds(src, L)] * D_TIME
            for j in range(D_TIME):
                v = plsc.load_gather(time_tbl, [tidx + j])
                plsc.store_scatter(buf, [cols[D_DRV + D_WEEK + j], rows], v)
            dvals = dist_v[pl.ds(src, L)]
            plsc.store_scatter(buf, [cols[D_OUT - 1], rows], dvals)
        gathers[r].wait()
        for g in range(GPC):
            rows = g * L + iota
            src = r * CHUNK + g * L
            dv = drv_idx[pl.ds(src, L)]
            off = (dv & (RPS - 1)) * D_DRV
            for j in range(D_DRV):
                v = plsc.load_gather(g_buf, [src + iota, off + j])
                plsc.store_scatter(buf, [cols[j], rows], v)
        out_writes[r % 2] = pltpu.async_copy(
            buf, out.at[:, pl.ds(base + r * CHUNK, CHUNK)], osems[r % 2])
    for w in out_writes:
        if w is not None:
            w.wait()


def kernel(driverID, weekID, timeID, dist, W_driver, W_week, W_time):
    wt_pad = jnp.pad(W_driver.T, ((0, 0), (0, V_PAD - V_DRV)))
    out_t, _ = _embed_concat(driverID.astype(jnp.int32),
                             weekID.astype(jnp.int32),
                             timeID.astype(jnp.int32),
                             dist.astype(jnp.float32),
                             wt_pad, W_week.reshape(-1), W_time.reshape(-1))
    return out_t.T


# speedup vs baseline: 1.3278x; 1.3278x over previous
import functools

import jax
import jax.numpy as jnp
from jax import lax
from jax.experimental import pallas as pl
from jax.experimental.pallas import tpu as pltpu
from jax.experimental.pallas import tpu_sc as plsc

B = 16384
D_DRV, D_WEEK, D_TIME = 16, 3, 8
D_OUT = D_DRV + D_WEEK + D_TIME + 1
V_DRV, V_WEEK, V_TIME = 24000, 7, 1440
SUP = 128
RPS = SUP // D_DRV
NC, NS = 2, 16
NW = NC * NS
BPW = B // NW
CHUNK = 128
NCHUNK = BPW // CHUNK
L = 16
GPC = CHUNK // L
SLABS_PER_TILE = 12
NSLAB = NS * SLABS_PER_TILE
V_PAD = NSLAB * CHUNK
SUP_PER_SC = V_PAD // RPS

_mesh = plsc.VectorSubcoreMesh(core_axis_name="c", subcore_axis_name="s")


@functools.partial(
    pl.kernel,
    mesh=_mesh,
    out_type=(jax.ShapeDtypeStruct((D_OUT, B), jnp.float32),
              jax.ShapeDtypeStruct((NC * SUP_PER_SC, SUP), jnp.float32)),
    compiler_params=pltpu.CompilerParams(needs_layout_passes=False),
    scratch_types=[
        pltpu.VMEM((BPW,), jnp.int32),
        pltpu.VMEM((BPW,), jnp.int32),
        pltpu.VMEM((BPW,), jnp.int32),
        pltpu.VMEM((BPW,), jnp.int32),
        pltpu.VMEM((BPW,), jnp.float32),
        pltpu.VMEM((BPW, SUP), jnp.float32),
        pltpu.VMEM((V_WEEK * D_WEEK,), jnp.float32),
        pltpu.VMEM((V_TIME * D_TIME,), jnp.float32),
        pltpu.VMEM((2, D_OUT, CHUNK), jnp.float32),
        pltpu.VMEM((2, D_DRV, CHUNK), jnp.float32),
        pltpu.VMEM((2, CHUNK // RPS, SUP), jnp.float32),
        [pltpu.SemaphoreType.DMA] * NCHUNK,
        [pltpu.SemaphoreType.DMA] * 2,
        pltpu.SemaphoreType.DMA,
        [pltpu.SemaphoreType.DMA] * 2,
        [pltpu.SemaphoreType.DMA] * 2,
    ],
)
def _embed_concat(drv_id, week_id, time_id, dist, wt_pad, w_week, w_time,
                  out, tbl, drv_idx, drv_sup, week_idx, time_idx, dist_v,
                  g_buf, week_tbl, time_tbl, out_buf, slab, srow,
                  gsems, osems, ssem, rsems, wsems):
    cc = lax.axis_index("c")
    sid = lax.axis_index("s")
    wid = sid * NC + cc
    base = wid * BPW
    sc_row0 = cc * SUP_PER_SC

    stagers = [
        pltpu.async_copy(week_id.at[pl.ds(base, BPW)], week_idx, ssem),
        pltpu.async_copy(time_id.at[pl.ds(base, BPW)], time_idx, ssem),
        pltpu.async_copy(dist.at[pl.ds(base, BPW)], dist_v, ssem),
        pltpu.async_copy(w_week, week_tbl, ssem),
        pltpu.async_copy(w_time, time_tbl, ssem),
    ]
    pltpu.sync_copy(drv_id.at[pl.ds(base, BPW)], drv_idx)

    iota = lax.iota(jnp.int32, L)
    cols = [jnp.full((L,), c, jnp.int32) for c in range(D_OUT)]

    @plsc.parallel_loop(0, BPW // L, unroll=4)
    def sup_body(g):
        v = drv_idx[pl.ds(g * L, L)]
        drv_sup[pl.ds(g * L, L)] = lax.shift_right_logical(v, 3) + sc_row0

    def slab_read(i, buf):
        b = sid * SLABS_PER_TILE + i
        col0 = pl.multiple_of(b * CHUNK, CHUNK)
        return pltpu.async_copy(wt_pad.at[:, pl.ds(col0, CHUNK)],
                                slab.at[buf], rsems[buf])

    reads = [slab_read(0, 0), slab_read(1, 1)]
    writes = [None, None]
    for i in range(SLABS_PER_TILE):
        bi = i % 2
        reads[bi].wait()
        if writes[bi] is not None:
            writes[bi].wait()

        @plsc.parallel_loop(0, CHUNK // RPS, unroll=4)
        def rep_body(k):
            for j in range(RPS):
                v = plsc.load_gather(slab.at[bi], [iota, cols[0] + (k * RPS + j)])
                srow[bi, k, pl.ds(j * D_DRV, D_DRV)] = v
        b = sid * SLABS_PER_TILE + i
        row0 = pl.multiple_of(sc_row0 + b * (CHUNK // RPS), CHUNK // RPS)
        writes[bi] = pltpu.async_copy(
            srow.at[bi], tbl.at[pl.ds(row0, CHUNK // RPS)], wsems[bi])
        if i + 2 < SLABS_PER_TILE:
            reads[bi] = slab_read(i + 2, bi)
    for w in writes:
        w.wait()
    plsc.subcore_barrier()

    gathers = [
        pltpu.async_copy(tbl.at[drv_sup.at[pl.ds(j * CHUNK, CHUNK)]],
                         g_buf.at[pl.ds(j * CHUNK, CHUNK)], gsems[j])
        for j in range(NCHUNK)
    ]
    for s in stagers:
        s.wait()

    out_writes = [None, None]
    for r in range(NCHUNK):
        buf = out_buf.at[r % 2]
        if out_writes[r % 2] is not None:
            out_writes[r % 2].wait()
        @plsc.parallel_loop(0, GPC, unroll=2)
        def small_body(g):
            rows = g * L + iota
            src = r * CHUNK + g * L
            widx = week_idx[pl.ds(src, L)] * D_WEEK
            for j in range(D_WEEK):
                v = plsc.load_gather(week_tbl, [widx + j])
                plsc.store_scatter(buf, [cols[D_DRV + j], rows], v)
            tidx = time_idx[pl.ds(src, L)] * D_TIME
            for j in range(D_TIME):
                v = plsc.load_gather(time_tbl, [tidx + j])
                plsc.store_scatter(buf, [cols[D_DRV + D_WEEK + j], rows], v)
            dvals = dist_v[pl.ds(src, L)]
            plsc.store_scatter(buf, [cols[D_OUT - 1], rows], dvals)
        gathers[r].wait()

        @plsc.parallel_loop(0, GPC, unroll=2)
        def drv_body(g):
            rows = g * L + iota
            src = r * CHUNK + g * L
            dv = drv_idx[pl.ds(src, L)]
            off = (dv & (RPS - 1)) * D_DRV
            for j in range(D_DRV):
                v = plsc.load_gather(g_buf, [src + iota, off + j])
                plsc.store_scatter(buf, [cols[j], rows], v)
        out_writes[r % 2] = pltpu.async_copy(
            buf, out.at[:, pl.ds(base + r * CHUNK, CHUNK)], osems[r % 2])
    for w in out_writes:
        if w is not None:
            w.wait()


def kernel(driverID, weekID, timeID, dist, W_driver, W_week, W_time):
    wt_pad = jnp.pad(W_driver.T, ((0, 0), (0, V_PAD - V_DRV)))
    out_t, _ = _embed_concat(driverID.astype(jnp.int32),
                             weekID.astype(jnp.int32),
                             timeID.astype(jnp.int32),
                             dist.astype(jnp.float32),
                             wt_pad, W_week.reshape(-1), W_time.reshape(-1))
    return out_t.T


# speedup vs baseline: 1.4192x; 1.0689x over previous
import functools

import jax
import jax.numpy as jnp
from jax import lax
from jax.experimental import pallas as pl
from jax.experimental.pallas import tpu as pltpu
from jax.experimental.pallas import tpu_sc as plsc

B = 16384
D_DRV, D_WEEK, D_TIME = 16, 3, 8
D_OUT = D_DRV + D_WEEK + D_TIME + 1
V_DRV, V_WEEK, V_TIME = 24000, 7, 1440
SUP = 128
RPS = SUP // D_DRV
NC, NS = 2, 16
NW = NC * NS
BPW = B // NW
CHUNK = 128
NCHUNK = BPW // CHUNK
L = 16
GPC = CHUNK // L
SLABS_PER_TILE = 12
NSLAB = NS * SLABS_PER_TILE
V_PAD = NSLAB * CHUNK
SUP_PER_SC = V_PAD // RPS

_mesh = plsc.VectorSubcoreMesh(core_axis_name="c", subcore_axis_name="s")


@functools.partial(
    pl.kernel,
    mesh=_mesh,
    out_type=(jax.ShapeDtypeStruct((D_OUT, B), jnp.float32),
              jax.ShapeDtypeStruct((NC * SUP_PER_SC, SUP), jnp.float32)),
    compiler_params=pltpu.CompilerParams(needs_layout_passes=False),
    scratch_types=[
        pltpu.VMEM((BPW,), jnp.int32),
        pltpu.VMEM((BPW,), jnp.int32),
        pltpu.VMEM((BPW,), jnp.int32),
        pltpu.VMEM((BPW,), jnp.int32),
        pltpu.VMEM((BPW,), jnp.float32),
        pltpu.VMEM((BPW, SUP), jnp.float32),
        pltpu.VMEM((D_WEEK, V_WEEK), jnp.float32),
        pltpu.VMEM((D_TIME, V_TIME), jnp.float32),
        pltpu.VMEM((2, D_OUT, CHUNK), jnp.float32),
        pltpu.VMEM((2, D_DRV, CHUNK), jnp.float32),
        pltpu.VMEM((2, CHUNK // RPS, SUP), jnp.float32),
        [pltpu.SemaphoreType.DMA] * NCHUNK,
        [pltpu.SemaphoreType.DMA] * 2,
        pltpu.SemaphoreType.DMA,
        [pltpu.SemaphoreType.DMA] * 2,
        [pltpu.SemaphoreType.DMA] * 2,
    ],
)
def _embed_concat(drv_id, week_id, time_id, dist, wt_pad, w_week, w_time,
                  out, tbl, drv_idx, drv_sup, week_idx, time_idx, dist_v,
                  g_buf, week_tbl, time_tbl, out_buf, slab, srow,
                  gsems, osems, ssem, rsems, wsems):
    cc = lax.axis_index("c")
    sid = lax.axis_index("s")
    wid = sid * NC + cc
    base = wid * BPW
    sc_row0 = cc * SUP_PER_SC

    stagers = [
        pltpu.async_copy(week_id.at[pl.ds(base, BPW)], week_idx, ssem),
        pltpu.async_copy(time_id.at[pl.ds(base, BPW)], time_idx, ssem),
        pltpu.async_copy(dist.at[pl.ds(base, BPW)], dist_v, ssem),
        pltpu.async_copy(w_week, week_tbl, ssem),
        pltpu.async_copy(w_time, time_tbl, ssem),
    ]
    pltpu.sync_copy(drv_id.at[pl.ds(base, BPW)], drv_idx)

    iota = lax.iota(jnp.int32, L)
    cols = [jnp.full((L,), c, jnp.int32) for c in range(D_OUT)]

    @plsc.parallel_loop(0, BPW // L, unroll=4)
    def sup_body(g):
        v = drv_idx[pl.ds(g * L, L)]
        drv_sup[pl.ds(g * L, L)] = lax.shift_right_logical(v, 3) + sc_row0

    def slab_read(i, buf):
        b = sid * SLABS_PER_TILE + i
        col0 = pl.multiple_of(b * CHUNK, CHUNK)
        return pltpu.async_copy(wt_pad.at[:, pl.ds(col0, CHUNK)],
                                slab.at[buf], rsems[buf])

    reads = [slab_read(0, 0), slab_read(1, 1)]
    writes = [None, None]
    for i in range(SLABS_PER_TILE):
        bi = i % 2
        reads[bi].wait()
        if writes[bi] is not None:
            writes[bi].wait()

        @plsc.parallel_loop(0, CHUNK // RPS, unroll=4)
        def rep_body(k):
            for j in range(RPS):
                v = plsc.load_gather(slab.at[bi], [iota, cols[0] + (k * RPS + j)])
                srow[bi, k, pl.ds(j * D_DRV, D_DRV)] = v
        b = sid * SLABS_PER_TILE + i
        row0 = pl.multiple_of(sc_row0 + b * (CHUNK // RPS), CHUNK // RPS)
        writes[bi] = pltpu.async_copy(
            srow.at[bi], tbl.at[pl.ds(row0, CHUNK // RPS)], wsems[bi])
        if i + 2 < SLABS_PER_TILE:
            reads[bi] = slab_read(i + 2, bi)
    for w in writes:
        w.wait()
    plsc.subcore_barrier()

    gathers = [
        pltpu.async_copy(tbl.at[drv_sup.at[pl.ds(j * CHUNK, CHUNK)]],
                         g_buf.at[pl.ds(j * CHUNK, CHUNK)], gsems[j])
        for j in range(NCHUNK)
    ]
    for s in stagers:
        s.wait()

    out_writes = [None, None]
    for r in range(NCHUNK):
        buf = out_buf.at[r % 2]
        if out_writes[r % 2] is not None:
            out_writes[r % 2].wait()
        @plsc.parallel_loop(0, GPC, unroll=4)
        def small_body(g):
            rows = g * L + iota
            src = r * CHUNK + g * L
            widx = week_idx[pl.ds(src, L)]
            for j in range(D_WEEK):
                v = plsc.load_gather(week_tbl, [cols[j], widx])
                plsc.store_scatter(buf, [cols[D_DRV + j], rows], v)
            tidx = time_idx[pl.ds(src, L)]
            for j in range(D_TIME):
                v = plsc.load_gather(time_tbl, [cols[j], tidx])
                plsc.store_scatter(buf, [cols[D_DRV + D_WEEK + j], rows], v)
            dvals = dist_v[pl.ds(src, L)]
            plsc.store_scatter(buf, [cols[D_OUT - 1], rows], dvals)
        gathers[r].wait()

        @plsc.parallel_loop(0, GPC, unroll=4)
        def drv_body(g):
            rows = g * L + iota
            src = r * CHUNK + g * L
            dv = drv_idx[pl.ds(src, L)]
            off = (dv & (RPS - 1)) * D_DRV
            for j in range(D_DRV):
                v = plsc.load_gather(g_buf, [src + iota, off + j])
                plsc.store_scatter(buf, [cols[j], rows], v)
        out_writes[r % 2] = pltpu.async_copy(
            buf, out.at[:, pl.ds(base + r * CHUNK, CHUNK)], osems[r % 2])
    for w in out_writes:
        if w is not None:
            w.wait()


def kernel(driverID, weekID, timeID, dist, W_driver, W_week, W_time):
    wt_pad = jnp.pad(W_driver.T, ((0, 0), (0, V_PAD - V_DRV)))
    out_t, _ = _embed_concat(driverID.astype(jnp.int32),
                             weekID.astype(jnp.int32),
                             timeID.astype(jnp.int32),
                             dist.astype(jnp.float32),
                             wt_pad, W_week.T, W_time.T)
    return out_t.T
